# baseline (speedup 1.0000x reference)
import functools
import math

import jax
import jax.numpy as jnp
from jax import lax
from jax.experimental import pallas as pl
from jax.experimental.pallas import tpu as pltpu
from jax.experimental.pallas import tpu_sc as plsc

D = 128
SCALE = math.sqrt(float(D))

NC = 2
NS = 16
NW = NC * NS

CHUNK = 128
NBUF = 5


def _scale_body(lut_ref, out_ref):
    out_ref[...] = lut_ref[...] * SCALE


def _scale_lut(lut):
    rows = lut.shape[0]
    block = 1000
    return pl.pallas_call(
        _scale_body,
        out_shape=jax.ShapeDtypeStruct(lut.shape, lut.dtype),
        grid=(rows // block,),
        in_specs=[pl.BlockSpec((block, D), lambda i: (i, 0))],
        out_specs=pl.BlockSpec((block, D), lambda i: (i, 0)),
    )(lut)


def _make_gather(B):
    assert B % (NW * CHUNK * NBUF) == 0
    b_per_w = B // NW
    n_groups = b_per_w // (CHUNK * NBUF)
    mesh = plsc.VectorSubcoreMesh(core_axis_name="c", subcore_axis_name="s")

    @functools.partial(
        pl.kernel,
        mesh=mesh,
        out_type=jax.ShapeDtypeStruct((B, D), jnp.float32),
        scratch_types=[
            pltpu.VMEM((NBUF, CHUNK), jnp.int32),
            pltpu.VMEM((NBUF, CHUNK, D), jnp.float32),
            [pltpu.SemaphoreType.DMA] * NBUF,
            [pltpu.SemaphoreType.DMA] * NBUF,
            [pltpu.SemaphoreType.DMA] * NBUF,
        ],
    )
    def gather(table_hbm, idx_hbm, out_hbm, idx_v, rows_v, sem_i, sem_g, sem_o):
        wid = lax.axis_index("s") * NC + lax.axis_index("c")
        base = wid * b_per_w

        for b in range(NBUF):
            pltpu.async_copy(
                idx_hbm.at[pl.ds(base + b * CHUNK, CHUNK)], idx_v.at[b], sem_i[b]
            )

        def group(g, carry):
            goff = base + g * (CHUNK * NBUF)
            for b in range(NBUF):

                @pl.when(g > 0)
                def _():
                    pltpu.make_async_copy(
                        rows_v.at[b], out_hbm.at[pl.ds(goff + b * CHUNK, CHUNK)],
                        sem_o[b],
                    ).wait()

                pltpu.make_async_copy(
                    idx_hbm.at[pl.ds(goff + b * CHUNK, CHUNK)], idx_v.at[b],
                    sem_i[b],
                ).wait()
                pltpu.async_copy(table_hbm.at[idx_v.at[b]], rows_v.at[b], sem_g[b])

            for b in range(NBUF):
                pltpu.make_async_copy(
                    table_hbm.at[idx_v.at[b]], rows_v.at[b], sem_g[b]
                ).wait()
                pltpu.async_copy(
                    rows_v.at[b], out_hbm.at[pl.ds(goff + b * CHUNK, CHUNK)],
                    sem_o[b],
                )

                @pl.when(g < n_groups - 1)
                def _():
                    pltpu.async_copy(
                        idx_hbm.at[pl.ds(goff + NBUF * CHUNK + b * CHUNK, CHUNK)],
                        idx_v.at[b],
                        sem_i[b],
                    )

            return carry

        lax.fori_loop(0, n_groups, group, 0)

        last = base + (n_groups - 1) * (CHUNK * NBUF)
        for b in range(NBUF):
            pltpu.make_async_copy(
                rows_v.at[b], out_hbm.at[pl.ds(last + b * CHUNK, CHUNK)], sem_o[b]
            ).wait()

    return gather


def kernel(x, lut):
    bt = x.shape[0] * x.shape[1]
    scaled = _scale_lut(lut)
    flat = x.reshape(bt)
    out = _make_gather(bt)(scaled, flat)
    return out.reshape(x.shape[0], x.shape[1], D)


# speedup vs baseline: 1.2172x; 1.2172x over previous
import functools
import math

import jax
import jax.numpy as jnp
from jax import lax
from jax.experimental import pallas as pl
from jax.experimental.pallas import tpu as pltpu
from jax.experimental.pallas import tpu_sc as plsc

D = 128
SCALE = math.sqrt(float(D))

NC = 2
NS = 16
NW = NC * NS

CHUNK = 128
NBUF = 5


def _make_gather(B):
    assert B % (NW * CHUNK * NBUF) == 0
    b_per_w = B // NW
    n_groups = b_per_w // (CHUNK * NBUF)
    mesh = plsc.VectorSubcoreMesh(core_axis_name="c", subcore_axis_name="s")

    @functools.partial(
        pl.kernel,
        mesh=mesh,
        out_type=jax.ShapeDtypeStruct((B, D), jnp.float32),
        scratch_types=[
            pltpu.VMEM((NBUF, CHUNK), jnp.int32),
            pltpu.VMEM((NBUF, CHUNK, D), jnp.float32),
            [pltpu.SemaphoreType.DMA] * NBUF,
            [pltpu.SemaphoreType.DMA] * NBUF,
            [pltpu.SemaphoreType.DMA] * NBUF,
        ],
    )
    def gather(table_hbm, idx_hbm, out_hbm, idx_v, rows_v, sem_i, sem_g, sem_o):
        wid = lax.axis_index("s") * NC + lax.axis_index("c")
        base = wid * b_per_w

        for b in range(NBUF):
            pltpu.async_copy(
                idx_hbm.at[pl.ds(base + b * CHUNK, CHUNK)], idx_v.at[b], sem_i[b]
            )

        def group(g, carry):
            goff = base + g * (CHUNK * NBUF)
            for b in range(NBUF):

                @pl.when(g > 0)
                def _():
                    pltpu.make_async_copy(
                        rows_v.at[b], out_hbm.at[pl.ds(goff + b * CHUNK, CHUNK)],
                        sem_o[b],
                    ).wait()

                pltpu.make_async_copy(
                    idx_hbm.at[pl.ds(goff + b * CHUNK, CHUNK)], idx_v.at[b],
                    sem_i[b],
                ).wait()
                pltpu.async_copy(table_hbm.at[idx_v.at[b]], rows_v.at[b], sem_g[b])

            for b in range(NBUF):
                pltpu.make_async_copy(
                    table_hbm.at[idx_v.at[b]], rows_v.at[b], sem_g[b]
                ).wait()

                @plsc.parallel_loop(0, CHUNK, step=1, unroll=4)
                def _(r):
                    for j in range(D // 16):
                        sl = pl.ds(j * 16, 16)
                        rows_v[b, r, sl] = rows_v[b, r, sl] * SCALE

                pltpu.async_copy(
                    rows_v.at[b], out_hbm.at[pl.ds(goff + b * CHUNK, CHUNK)],
                    sem_o[b],
                )

                @pl.when(g < n_groups - 1)
                def _():
                    pltpu.async_copy(
                        idx_hbm.at[pl.ds(goff + NBUF * CHUNK + b * CHUNK, CHUNK)],
                        idx_v.at[b],
                        sem_i[b],
                    )

            return carry

        lax.fori_loop(0, n_groups, group, 0)

        last = base + (n_groups - 1) * (CHUNK * NBUF)
        for b in range(NBUF):
            pltpu.make_async_copy(
                rows_v.at[b], out_hbm.at[pl.ds(last + b * CHUNK, CHUNK)], sem_o[b]
            ).wait()

    return gather


def kernel(x, lut):
    bt = x.shape[0] * x.shape[1]
    flat = x.reshape(bt)
    out = _make_gather(bt)(lut, flat)
    return out.reshape(x.shape[0], x.shape[1], D)
